# baseline (speedup 1.0000x reference)
import functools

import jax
import jax.numpy as jnp
from jax import lax
from jax.experimental import pallas as pl
from jax.experimental.pallas import tpu as pltpu
from jax.experimental.pallas import tpu_sc as plsc

B, T, F, H, HEADS = 16, 20, 64, 128, 4
OUT = H // HEADS
E0 = 512
FG = 8
TG = 4


def _adj_sc_kernel(ei_hbm, out_hbm, ei_v, zero_v, ones_v, kidx, acc_sh):
    cid = lax.axis_index("c")
    sid = lax.axis_index("s")

    @pl.when(jnp.logical_and(cid == 0, sid == 0))
    def _():
        pltpu.sync_copy(ei_hbm, ei_v)
        for c in range(E0 // 16):
            src = ei_v[0, pl.ds(c * 16, 16)]
            dst = ei_v[1, pl.ds(c * 16, 16)]
            kidx[pl.ds(c * 16, 16)] = dst * F + src
        for c in range(F * F // 16):
            zero_v[pl.ds(c * 16, 16)] = jnp.zeros((16,), jnp.float32)
        for c in range(E0 // 16):
            ones_v[pl.ds(c * 16, 16)] = jnp.ones((16,), jnp.float32)
        pltpu.sync_copy(zero_v, acc_sh)
        for c in range(E0 // 128):
            pltpu.sync_copy(ones_v.at[pl.ds(c * 128, 128)],
                            acc_sh.at[kidx.at[pl.ds(c * 128, 128)]],
                            add=True)
        pltpu.sync_copy(acc_sh, zero_v)
        pltpu.sync_copy(zero_v, out_hbm)


def _adj_counts(ei):
    k = functools.partial(
        pl.kernel,
        out_type=jax.ShapeDtypeStruct((F * F,), jnp.float32),
        mesh=plsc.VectorSubcoreMesh(core_axis_name="c", subcore_axis_name="s"),
        scratch_types=[
            pltpu.VMEM((2, E0), jnp.int32),
            pltpu.VMEM((F * F,), jnp.float32),
            pltpu.VMEM((E0,), jnp.float32),
            pltpu.VMEM((E0,), jnp.int32),
            pltpu.VMEM_SHARED((F * F,), jnp.float32),
        ],
    )(_adj_sc_kernel)
    return k(ei)


NP1 = T // TG


def _fused_kernel(hist_ref, histl_ref, a_ref, M_ref, foff_ref,
                  prt_ref, qrt_ref, pl_ref, ql_ref, bgat_ref,
                  Wih_ref, Whh_ref, bfold_ref, bhhn_ref, Wdec_ref,
                  hid_ref, ans_ref, gnn_s):
    i = pl.program_id(0)

    @pl.when(i < NP1)
    def _gat():
        r_i = jax.lax.broadcasted_iota(jnp.int32, (F, F), 0)
        c_i = jax.lax.broadcasted_iota(jnp.int32, (F, F), 1)
        A = a_ref[...] + (r_i == c_i).astype(jnp.float32)
        lnA = jnp.where(A > 0.0, jnp.log(A), -1e30)
        ones_n = jnp.ones((F, F), jnp.float32)
        for tg in range(TG):
            hist = hist_ref[tg]
            histl = histl_ref[tg]
            feat = hist * M_ref[...][None] + foff_ref[...][None]

            outs = []
            for h in range(HEADS):
                fh = feat[:, :, h * OUT:(h + 1) * OUT]
                er_col = hist * prt_ref[h][None] + qrt_ref[h][None]
                el_row = histl * pl_ref[h][None, None] + ql_ref[h][None, None]
                e = er_col + el_row
                e = jnp.maximum(e, 0.2 * e)
                e = e + lnA[None]
                m = e.max(axis=-1, keepdims=True)
                ee = jnp.exp(e - m)
                s_b = jax.lax.dot_general(
                    ee, ones_n, (((2,), (0,)), ((), ())),
                    preferred_element_type=jnp.float32)
                agg = jax.lax.dot_general(
                    ee, fh, (((2,), (1,)), ((0,), (0,))),
                    preferred_element_type=jnp.float32)
                outs.append(agg / s_b[:, :, :OUT])
            rst = jnp.concatenate(outs, axis=-1)
            rst = rst + bgat_ref[...][None, None]
            gnn_s[:, i * TG + tg, :, :] = rst.transpose(1, 0, 2)

    @pl.when(i >= NP1)
    def _gru():
        fbase = (i - NP1) * FG
        gis = []
        hs = []
        for fl in range(FG):
            x_f = gnn_s[fbase + fl]
            gi = jax.lax.dot_general(x_f, Wih_ref[fl],
                                     (((2,), (1,)), ((), ())),
                                     preferred_element_type=jnp.float32)
            gis.append(gi + bfold_ref[fl, 0][None, None])
            hs.append(jnp.zeros((B, H), jnp.float32))
        outs = [[] for _ in range(FG)]

        for t in range(T):
            for fl in range(FG):
                h = hs[fl]
                gi = gis[fl][t]
                gh = jax.lax.dot_general(h, Whh_ref[fl],
                                         (((1,), (1,)), ((), ())),
                                         preferred_element_type=jnp.float32)
                rz = 0.5 + 0.5 * jnp.tanh(
                    0.5 * (gi[:, :2 * H] + gh[:, :2 * H]))
                r = rz[:, :H]
                z = rz[:, H:]
                n = jnp.tanh(gi[:, 2 * H:]
                             + r * (gh[:, 2 * H:] + bhhn_ref[fl, 0][None]))
                h = n + z * (h - n)
                hs[fl] = h
                hid_ref[:, t, fl, :] = h
                outs[fl].append(h)

        for fl in range(FG):
            hst = jnp.stack(outs[fl], axis=0)
            ans_ref[fl] = (hst * Wdec_ref[fl, 0][None, None]).sum(-1)


def kernel(history_stack, edge_index, W_enc, b_enc, W_gat, attn_l, attn_r,
           b_gat, W_ih, W_hh, b_ih, b_hh, W_dec, b_dec):
    hist_t = history_stack.transpose(1, 0, 2)
    hist_e = hist_t[..., None]
    hist_l = hist_t[:, :, None, :]
    ei = edge_index.astype(jnp.int32)
    b_fold = b_ih + jnp.concatenate(
        [b_hh[:, :2 * H], jnp.zeros_like(b_hh[:, 2 * H:])], axis=1)
    bhh_n = b_hh[:, 2 * H:]
    feat_off = b_enc @ W_gat
    M_enc = W_enc @ W_gat
    Mh = M_enc.reshape(F, HEADS, OUT)
    Fh = feat_off.reshape(F, HEADS, OUT)
    Pl = jnp.einsum('fho,ho->hf', Mh, attn_l)
    ql = jnp.einsum('fho,ho->hf', Fh, attn_l)
    PrT = jnp.einsum('fho,ho->hf', Mh, attn_r)[:, :, None]
    qrT = jnp.einsum('fho,ho->hf', Fh, attn_r)[:, :, None]

    A_counts = _adj_counts(ei).reshape(F, F)

    def _p1(i):
        return jnp.minimum(i, NP1 - 1)

    def _p2(i):
        return jnp.maximum(i - NP1, 0)

    hid_perm, ans_perm = pl.pallas_call(
        _fused_kernel,
        grid=(NP1 + F // FG,),
        in_specs=[
            pl.BlockSpec((TG, B, F, 1), lambda i: (_p1(i), 0, 0, 0)),
            pl.BlockSpec((TG, B, 1, F), lambda i: (_p1(i), 0, 0, 0)),
            pl.BlockSpec((F, F), lambda i: (0, 0)),
            pl.BlockSpec((F, H), lambda i: (0, 0)),
            pl.BlockSpec((F, H), lambda i: (0, 0)),
            pl.BlockSpec((HEADS, F, 1), lambda i: (0, 0, 0)),
            pl.BlockSpec((HEADS, F, 1), lambda i: (0, 0, 0)),
            pl.BlockSpec((HEADS, F), lambda i: (0, 0)),
            pl.BlockSpec((HEADS, F), lambda i: (0, 0)),
            pl.BlockSpec((H,), lambda i: (0,)),
            pl.BlockSpec((FG, 3 * H, H), lambda i: (_p2(i), 0, 0)),
            pl.BlockSpec((FG, 3 * H, H), lambda i: (_p2(i), 0, 0)),
            pl.BlockSpec((FG, 1, 3 * H), lambda i: (_p2(i), 0, 0)),
            pl.BlockSpec((FG, 1, H), lambda i: (_p2(i), 0, 0)),
            pl.BlockSpec((FG, 1, H), lambda i: (_p2(i), 0, 0)),
        ],
        out_specs=[
            pl.BlockSpec((B, T, FG, H), lambda i: (0, 0, _p2(i), 0)),
            pl.BlockSpec((FG, T, B), lambda i: (_p2(i), 0, 0)),
        ],
        out_shape=[
            jax.ShapeDtypeStruct((B, T, F, H), jnp.float32),
            jax.ShapeDtypeStruct((F, T, B), jnp.float32),
        ],
        scratch_shapes=[pltpu.VMEM((F, T, B, H), jnp.float32)],
    )(hist_e, hist_l, A_counts, M_enc, feat_off, PrT, qrT, Pl, ql, b_gat,
      W_ih, W_hh, b_fold[:, None], bhh_n[:, None], W_dec[:, None])

    ans = ans_perm.transpose(2, 1, 0) + b_dec[None, None]
    return (ans, hid_perm)


# speedup vs baseline: 1.0062x; 1.0062x over previous
import functools

import jax
import jax.numpy as jnp
from jax import lax
from jax.experimental import pallas as pl
from jax.experimental.pallas import tpu as pltpu
from jax.experimental.pallas import tpu_sc as plsc

B, T, F, H, HEADS = 16, 20, 64, 128, 4
OUT = H // HEADS
E0 = 512
FG = 8
TG = 4


def _adj_sc_kernel(ei_hbm, out_hbm, ei_v, zero_v, ones_v, kidx, acc_sh):
    cid = lax.axis_index("c")
    sid = lax.axis_index("s")

    @pl.when(jnp.logical_and(cid == 0, sid == 0))
    def _():
        pltpu.sync_copy(ei_hbm, ei_v)
        for c in range(E0 // 16):
            src = ei_v[0, pl.ds(c * 16, 16)]
            dst = ei_v[1, pl.ds(c * 16, 16)]
            kidx[pl.ds(c * 16, 16)] = dst * F + src
        for c in range(F * F // 16):
            zero_v[pl.ds(c * 16, 16)] = jnp.zeros((16,), jnp.float32)
        for c in range(E0 // 16):
            ones_v[pl.ds(c * 16, 16)] = jnp.ones((16,), jnp.float32)
        pltpu.sync_copy(zero_v, acc_sh)
        for c in range(E0 // 128):
            pltpu.sync_copy(ones_v.at[pl.ds(c * 128, 128)],
                            acc_sh.at[kidx.at[pl.ds(c * 128, 128)]],
                            add=True)
        pltpu.sync_copy(acc_sh, zero_v)
        pltpu.sync_copy(zero_v, out_hbm)


def _adj_counts(ei):
    k = functools.partial(
        pl.kernel,
        out_type=jax.ShapeDtypeStruct((F * F,), jnp.float32),
        mesh=plsc.VectorSubcoreMesh(core_axis_name="c", subcore_axis_name="s"),
        scratch_types=[
            pltpu.VMEM((2, E0), jnp.int32),
            pltpu.VMEM((F * F,), jnp.float32),
            pltpu.VMEM((E0,), jnp.float32),
            pltpu.VMEM((E0,), jnp.int32),
            pltpu.VMEM_SHARED((F * F,), jnp.float32),
        ],
    )(_adj_sc_kernel)
    return k(ei)


def _gat_kernel(hist_ref, histl_ref, a_ref, M_ref, foff_ref,
                prt_ref, qrt_ref, pl_ref, ql_ref, bgat_ref, out_ref):
    r_i = jax.lax.broadcasted_iota(jnp.int32, (F, F), 0)
    c_i = jax.lax.broadcasted_iota(jnp.int32, (F, F), 1)
    A = a_ref[...] + (r_i == c_i).astype(jnp.float32)
    lnA = jnp.where(A > 0.0, jnp.log(A), -1e30)
    ones_n = jnp.ones((F, F), jnp.float32)
    for tg in range(TG):
        hist = hist_ref[tg]
        histl = histl_ref[tg]
        feat = hist * M_ref[...][None] + foff_ref[...][None]

        outs = []
        for h in range(HEADS):
            fh = feat[:, :, h * OUT:(h + 1) * OUT]
            er_col = hist * prt_ref[h][None] + qrt_ref[h][None]
            el_row = histl * pl_ref[h][None, None] + ql_ref[h][None, None]
            e = er_col + el_row
            e = jnp.maximum(e, 0.2 * e)
            e = e + lnA[None]
            m = e.max(axis=-1, keepdims=True)
            ee = jnp.exp(e - m)
            s_b = jax.lax.dot_general(ee, ones_n, (((2,), (0,)), ((), ())),
                                      preferred_element_type=jnp.float32)
            agg = jax.lax.dot_general(ee, fh, (((2,), (1,)), ((0,), (0,))),
                                      preferred_element_type=jnp.float32)
            outs.append(agg / s_b[:, :, :OUT])
        rst = jnp.concatenate(outs, axis=-1)
        rst = rst + bgat_ref[...][None, None]
        out_ref[:, tg, :, :] = rst.transpose(1, 0, 2)


def _gru_kernel(gnn_ref, Wih_ref, Whh_ref, bfold_ref, bhhn_ref,
                Wdec_ref, hid_ref, ans_ref):
    gis = []
    hs = []
    for fl in range(FG):
        x_f = gnn_ref[fl]
        gi = jax.lax.dot_general(x_f, Wih_ref[fl], (((2,), (1,)), ((), ())),
                                 preferred_element_type=jnp.float32)
        gis.append(gi + bfold_ref[fl, 0][None, None])
        hs.append(jnp.zeros((B, H), jnp.float32))
    outs = [[] for _ in range(FG)]

    for t in range(T):
        for fl in range(FG):
            h = hs[fl]
            gi = gis[fl][t]
            gh = jax.lax.dot_general(h, Whh_ref[fl], (((1,), (1,)), ((), ())),
                                     preferred_element_type=jnp.float32)
            rz = 0.5 + 0.5 * jnp.tanh(
                0.5 * (gi[:, :2 * H] + gh[:, :2 * H]))
            r = rz[:, :H]
            z = rz[:, H:]
            n = jnp.tanh(gi[:, 2 * H:]
                         + r * (gh[:, 2 * H:] + bhhn_ref[fl, 0][None]))
            h = n + z * (h - n)
            hs[fl] = h
            hid_ref[:, t, fl, :] = h
            outs[fl].append(h)

    for fl in range(FG):
        hst = jnp.stack(outs[fl], axis=0)
        ans_ref[fl] = (hst * Wdec_ref[fl, 0][None, None]).sum(-1)


def kernel(history_stack, edge_index, W_enc, b_enc, W_gat, attn_l, attn_r,
           b_gat, W_ih, W_hh, b_ih, b_hh, W_dec, b_dec):
    hist_t = history_stack.transpose(1, 0, 2)
    hist_e = hist_t[..., None]
    hist_l = hist_t[:, :, None, :]
    ei = edge_index.astype(jnp.int32)
    b_fold = b_ih + jnp.concatenate(
        [b_hh[:, :2 * H], jnp.zeros_like(b_hh[:, 2 * H:])], axis=1)
    bhh_n = b_hh[:, 2 * H:]
    feat_off = b_enc @ W_gat
    M_enc = W_enc @ W_gat
    Mh = M_enc.reshape(F, HEADS, OUT)
    Fh = feat_off.reshape(F, HEADS, OUT)
    Pl = jnp.einsum('fho,ho->hf', Mh, attn_l)
    ql = jnp.einsum('fho,ho->hf', Fh, attn_l)
    PrT = jnp.einsum('fho,ho->hf', Mh, attn_r)[:, :, None]
    qrT = jnp.einsum('fho,ho->hf', Fh, attn_r)[:, :, None]

    A_counts = _adj_counts(ei).reshape(F, F)

    gnn = pl.pallas_call(
        _gat_kernel,
        grid=(T // TG,),
        in_specs=[
            pl.BlockSpec((TG, B, F, 1), lambda t: (t, 0, 0, 0)),
            pl.BlockSpec((TG, B, 1, F), lambda t: (t, 0, 0, 0)),
            pl.BlockSpec((F, F), lambda t: (0, 0)),
            pl.BlockSpec((F, H), lambda t: (0, 0)),
            pl.BlockSpec((F, H), lambda t: (0, 0)),
            pl.BlockSpec((HEADS, F, 1), lambda t: (0, 0, 0)),
            pl.BlockSpec((HEADS, F, 1), lambda t: (0, 0, 0)),
            pl.BlockSpec((HEADS, F), lambda t: (0, 0)),
            pl.BlockSpec((HEADS, F), lambda t: (0, 0)),
            pl.BlockSpec((H,), lambda t: (0,)),
        ],
        out_specs=pl.BlockSpec((F, TG, B, H), lambda t: (0, t, 0, 0)),
        out_shape=jax.ShapeDtypeStruct((F, T, B, H), jnp.float32),
        compiler_params=pltpu.CompilerParams(
            dimension_semantics=("parallel",)),
    )(hist_e, hist_l, A_counts, M_enc, feat_off, PrT, qrT, Pl, ql, b_gat)

    hid_perm, ans_perm = pl.pallas_call(
        _gru_kernel,
        grid=(F // FG,),
        in_specs=[
            pl.BlockSpec((FG, T, B, H), lambda f: (f, 0, 0, 0)),
            pl.BlockSpec((FG, 3 * H, H), lambda f: (f, 0, 0)),
            pl.BlockSpec((FG, 3 * H, H), lambda f: (f, 0, 0)),
            pl.BlockSpec((FG, 1, 3 * H), lambda f: (f, 0, 0)),
            pl.BlockSpec((FG, 1, H), lambda f: (f, 0, 0)),
            pl.BlockSpec((FG, 1, H), lambda f: (f, 0, 0)),
        ],
        out_specs=[
            pl.BlockSpec((B, T, FG, H), lambda f: (0, 0, f, 0)),
            pl.BlockSpec((FG, T, B), lambda f: (f, 0, 0)),
        ],
        out_shape=[
            jax.ShapeDtypeStruct((B, T, F, H), jnp.float32),
            jax.ShapeDtypeStruct((F, T, B), jnp.float32),
        ],
        compiler_params=pltpu.CompilerParams(
            dimension_semantics=("parallel",)),
    )(gnn, W_ih, W_hh, b_fold[:, None], bhh_n[:, None], W_dec[:, None])

    ans = ans_perm.transpose(2, 1, 0) + b_dec[None, None]
    return (ans, hid_perm)
